# baseline (speedup 1.0000x reference)
import functools

import jax
import jax.numpy as jnp
from jax import lax
from jax.experimental import pallas as pl
from jax.experimental.pallas import tpu as pltpu
from jax.experimental.pallas import tpu_sc as plsc

EMB = 64
K = 128
NBUF = 2


@functools.lru_cache(maxsize=None)
def _build(n_rows: int, emb: int):
    info = plsc.get_sparse_core_info()
    nc, ns = info.num_cores, info.num_subcores
    nw = nc * ns
    assert n_rows % (nw * K) == 0
    chunks_per_worker = n_rows // (nw * K)

    mesh = plsc.VectorSubcoreMesh(core_axis_name="c", subcore_axis_name="s")
    out_t = jax.ShapeDtypeStruct((n_rows, emb), jnp.float32)

    @functools.partial(
        pl.kernel,
        mesh=mesh,
        out_type=(out_t, out_t),
        scratch_types=(
            pltpu.VMEM((chunks_per_worker, K), jnp.int32),
            pltpu.VMEM((K, emb), jnp.float32),
            pltpu.VMEM((K, emb), jnp.float32),
            pltpu.SemaphoreType.DMA,
            pltpu.SemaphoreType.DMA,
        ),
        compiler_params=pltpu.CompilerParams(use_tc_tiling_on_sc=False),
    )
    def gather2(table_hbm, idx_a_hbm, idx_b_hbm, out_a_hbm, out_b_hbm,
                idx_v, rows0, rows1, sem0, sem1):
        wid = lax.axis_index("s") * nc + lax.axis_index("c")
        chunk_base = wid * chunks_per_worker
        bufs = (rows0, rows1)
        sems = (sem0, sem1)

        for idx_hbm, out_hbm in ((idx_a_hbm, out_a_hbm), (idx_b_hbm, out_b_hbm)):
            pltpu.sync_copy(idx_hbm.at[pl.ds(chunk_base, chunks_per_worker), :],
                            idx_v)
            for b in range(NBUF):
                pltpu.async_copy(table_hbm.at[idx_v.at[b]], bufs[b], sems[b])

            def outer(g, _):
                for b in range(NBUF):
                    j = g * NBUF + b
                    pltpu.make_async_copy(
                        table_hbm.at[idx_v.at[j]], bufs[b], sems[b]).wait()
                    pltpu.sync_copy(
                        bufs[b],
                        out_hbm.at[pl.ds((chunk_base + j) * K, K), :])

                    @pl.when(j + NBUF < chunks_per_worker)
                    def _():
                        pltpu.async_copy(
                            table_hbm.at[idx_v.at[j + NBUF]], bufs[b], sems[b])
                return 0

            lax.fori_loop(0, chunks_per_worker // NBUF, outer, 0)

    return gather2


def kernel(table, inputs, support):
    bsz, seq = inputs.shape
    n_rows = bsz * seq
    idx_a = inputs.astype(jnp.int32).reshape(n_rows // K, K)
    idx_b = support.astype(jnp.int32).reshape(n_rows // K, K)
    fn = _build(n_rows, table.shape[1])
    out_a, out_b = fn(table, idx_a, idx_b)
    return (out_a.reshape(bsz, seq, table.shape[1]),
            out_b.reshape(bsz, seq, table.shape[1]))


# speedup vs baseline: 1.0312x; 1.0312x over previous
import functools

import jax
import jax.numpy as jnp
from jax import lax
from jax.experimental import pallas as pl
from jax.experimental.pallas import tpu as pltpu
from jax.experimental.pallas import tpu_sc as plsc

EMB = 64
K = 128
NBUF = 4
H = 2


@functools.lru_cache(maxsize=None)
def _build(n_rows: int, emb: int):
    info = plsc.get_sparse_core_info()
    nc, ns = info.num_cores, info.num_subcores
    nw = nc * ns
    assert n_rows % (nw * K) == 0
    chunks_per_worker = n_rows // (nw * K)
    n = chunks_per_worker

    mesh = plsc.VectorSubcoreMesh(core_axis_name="c", subcore_axis_name="s")
    out_t = jax.ShapeDtypeStruct((n_rows, emb), jnp.float32)

    @functools.partial(
        pl.kernel,
        mesh=mesh,
        out_type=(out_t, out_t),
        scratch_types=(
            pltpu.VMEM((n, K), jnp.int32),
            tuple(pltpu.VMEM((K, emb), jnp.float32) for _ in range(NBUF)),
            tuple(pltpu.SemaphoreType.DMA for _ in range(NBUF)),
            tuple(pltpu.SemaphoreType.DMA for _ in range(NBUF)),
        ),
        compiler_params=pltpu.CompilerParams(use_tc_tiling_on_sc=False),
    )
    def gather2(table_hbm, idx_a_hbm, idx_b_hbm, out_a_hbm, out_b_hbm,
                idx_v, bufs, gsems, wsems):
        wid = lax.axis_index("s") * nc + lax.axis_index("c")
        chunk_base = wid * n

        for idx_hbm, out_hbm in ((idx_a_hbm, out_a_hbm), (idx_b_hbm, out_b_hbm)):
            pltpu.sync_copy(idx_hbm.at[pl.ds(chunk_base, n), :], idx_v)
            for b in range(H):
                pltpu.async_copy(table_hbm.at[idx_v.at[b]], bufs[b], gsems[b])

            def outer(g, _):
                for b in range(NBUF):
                    j = g * NBUF + b
                    bi = (b + H) % NBUF

                    @pl.when(j + H < n)
                    def _():
                        @pl.when(j + H >= NBUF)
                        def _():
                            pltpu.make_async_copy(
                                bufs[bi],
                                out_hbm.at[pl.ds((chunk_base + j) * K, K), :],
                                wsems[bi]).wait()
                        pltpu.async_copy(
                            table_hbm.at[idx_v.at[j + H]], bufs[bi], gsems[bi])

                    pltpu.make_async_copy(
                        table_hbm.at[idx_v.at[j]], bufs[b], gsems[b]).wait()
                    pltpu.async_copy(
                        bufs[b],
                        out_hbm.at[pl.ds((chunk_base + j) * K, K), :],
                        wsems[b])
                return 0

            lax.fori_loop(0, n // NBUF, outer, 0)
            for b in range(NBUF):
                pltpu.make_async_copy(
                    bufs[b], out_hbm.at[pl.ds(chunk_base * K, K), :],
                    wsems[b]).wait()

    return gather2


def kernel(table, inputs, support):
    bsz, seq = inputs.shape
    n_rows = bsz * seq
    idx_a = inputs.astype(jnp.int32).reshape(n_rows // K, K)
    idx_b = support.astype(jnp.int32).reshape(n_rows // K, K)
    fn = _build(n_rows, table.shape[1])
    out_a, out_b = fn(table, idx_a, idx_b)
    return (out_a.reshape(bsz, seq, table.shape[1]),
            out_b.reshape(bsz, seq, table.shape[1]))
